# baseline (speedup 1.0000x reference)
import functools

import jax
import jax.numpy as jnp
from jax import lax
from jax.experimental import pallas as pl
from jax.experimental.pallas import tpu as pltpu
from jax.experimental.pallas import tpu_sc as plsc

N_ROWS = 100000
D = 128
NSEG = 64
G = 128
NC = 2
NS = 16
NW = NC * NS
ROWS_PER_W = N_ROWS // NW
N_BIG = (ROWS_PER_W - 8) // G
NBUF = 3
BC_COLS = 12544
BC_GRID = BC_COLS // 128


def _tc_bincount(bi_ref, cnt_ref, acc_ref):
    i = pl.program_id(0)

    @pl.when(i == 0)
    def _():
        acc_ref[...] = jnp.zeros((NSEG, D), jnp.float32)

    blk = bi_ref[...]
    seg = lax.broadcasted_iota(jnp.int32, (NSEG, D), 0)
    tot = jnp.zeros((NSEG, D), jnp.float32)
    for r in range(8):
        row = jnp.broadcast_to(blk[r:r + 1, :], (NSEG, D))
        tot = tot + (row == seg).astype(jnp.float32)
    acc_ref[...] += tot

    @pl.when(i == BC_GRID - 1)
    def _():
        cnt_ref[...] = acc_ref[...]


def _tc_finish(sums_ref, cnts_ref, out_ref):
    s = sums_ref[0] + sums_ref[1]
    c = jnp.sum(cnts_ref[...], axis=1, keepdims=True)
    out_ref[...] = s / jnp.maximum(c, 1.0)


def kernel(x, batch_index):
    bi = batch_index.astype(jnp.int32)
    mesh = plsc.VectorSubcoreMesh(core_axis_name="c", subcore_axis_name="s")

    @functools.partial(
        pl.kernel,
        mesh=mesh,
        out_type=jax.ShapeDtypeStruct((NC, NSEG, D), jnp.float32),
        scratch_types=(
            [pltpu.VMEM((G, D), jnp.float32) for _ in range(NBUF)]
            + [pltpu.VMEM((G,), jnp.int32) for _ in range(NBUF)]
            + [
                pltpu.VMEM((4, D), jnp.float32),
                pltpu.VMEM((4, D), jnp.float32),
                pltpu.VMEM((8, D), jnp.float32),
                pltpu.VMEM((8,), jnp.int32),
                pltpu.VMEM_SHARED((NS * NSEG, D), jnp.float32),
            ]
            + [pltpu.SemaphoreType.DMA for _ in range(3 * NBUF)]
        ),
    )
    def sc_seg(x_hbm, bi_hbm, sums_out,
               rows0, rows1, rows2, idx0, idx1, idx2,
               red_v, racc_v, rows8_v, idx8_v, stage_sh,
               gr0, gr1, gr2, gi0, gi1, gi2, ss0, ss1, ss2):
        rows_b = (rows0, rows1, rows2)
        idx_b = (idx0, idx1, idx2)
        sem_gr = (gr0, gr1, gr2)
        sem_gi = (gi0, gi1, gi2)
        sem_s = (ss0, ss1, ss2)

        c = lax.axis_index("c")
        s = lax.axis_index("s")
        wid = c * NS + s

        zero16 = jnp.zeros((16,), jnp.float32)

        for r in range(4):
            for j in range(D // 16):
                red_v[r, pl.ds(j * 16, 16)] = zero16
        for k in range(NS):
            pltpu.sync_copy(red_v, stage_sh.at[pl.ds(s * NSEG + k * 4, 4)])
        my_acc = stage_sh.at[pl.ds(s * NSEG, NSEG)]

        start = (wid * ROWS_PER_W) & -8
        end = jnp.where(wid == NW - 1, N_ROWS, ((wid + 1) * ROWS_PER_W) & -8)
        tail0 = start + N_BIG * G
        n_tail = (end - tail0) // 8

        gathers = {}
        scatters = {}

        def issue_gather(g):
            b = g % NBUF
            off = pl.multiple_of(start + g * G, 8)
            gathers[g] = (
                pltpu.async_copy(x_hbm.at[pl.ds(off, G)], rows_b[b], sem_gr[b]),
                pltpu.async_copy(bi_hbm.at[pl.ds(off, G)], idx_b[b], sem_gi[b]),
            )

        issue_gather(0)
        issue_gather(1)
        for g in range(N_BIG):
            b = g % NBUF
            for d in gathers.pop(g):
                d.wait()
            scatters[g] = pltpu.async_copy(
                rows_b[b], my_acc.at[idx_b[b]], sem_s[b], add=True)
            if g + 2 < N_BIG:
                if g >= 1:
                    scatters.pop(g - 1).wait()
                issue_gather(g + 2)
        for g in sorted(scatters):
            scatters.pop(g).wait()

        def tail_body(t, carry):
            off = pl.multiple_of(tail0 + t * 8, 8)
            pltpu.sync_copy(x_hbm.at[pl.ds(off, 8)], rows8_v)
            pltpu.sync_copy(bi_hbm.at[pl.ds(off, 8)], idx8_v)
            pltpu.sync_copy(rows8_v, my_acc.at[idx8_v], add=True)
            return carry

        lax.fori_loop(0, n_tail, tail_body, None)

        plsc.subcore_barrier()

        for j in range(D // 16):
            for r in range(4):
                racc_v[r, pl.ds(j * 16, 16)] = zero16
        for k in range(NS):
            pltpu.sync_copy(
                stage_sh.at[pl.ds(k * NSEG + s * 4, 4)], red_v)
            for r in range(4):
                for j in range(D // 16):
                    racc_v[r, pl.ds(j * 16, 16)] = (
                        racc_v[r, pl.ds(j * 16, 16)]
                        + red_v[r, pl.ds(j * 16, 16)])
        pltpu.sync_copy(racc_v, sums_out.at[c, pl.ds(s * 4, 4)])

    sums = sc_seg(x, bi)

    bi2d = jnp.pad(bi, (0, 8 * BC_COLS - N_ROWS),
                   constant_values=NSEG).reshape(8, BC_COLS)
    cnts = pl.pallas_call(
        _tc_bincount,
        grid=(BC_GRID,),
        in_specs=[pl.BlockSpec((8, 128), lambda i: (0, i))],
        out_specs=pl.BlockSpec((NSEG, D), lambda i: (0, 0)),
        out_shape=jax.ShapeDtypeStruct((NSEG, D), jnp.float32),
        scratch_shapes=[pltpu.VMEM((NSEG, D), jnp.float32)],
    )(bi2d)

    out = pl.pallas_call(
        _tc_finish,
        out_shape=jax.ShapeDtypeStruct((NSEG, D), jnp.float32),
    )(sums, cnts)
    return out


# speedup vs baseline: 1.1598x; 1.1598x over previous
import functools

import jax
import jax.numpy as jnp
from jax import lax
from jax.experimental import pallas as pl
from jax.experimental.pallas import tpu as pltpu
from jax.experimental.pallas import tpu_sc as plsc

N_ROWS = 100000
D = 128
NSEG = 64
G = 128
NC = 2
NS = 16
NW = NC * NS
ROWS_PER_W = N_ROWS // NW
N_BIG = (ROWS_PER_W - 8) // G
NBUF = 6
AHEAD = NBUF - 2
BC_COLS = 12544
BC_GRID = BC_COLS // 128


def _tc_bincount(bi_ref, cnt_ref, acc_ref):
    i = pl.program_id(0)

    @pl.when(i == 0)
    def _():
        acc_ref[...] = jnp.zeros((NSEG, D), jnp.float32)

    blk = bi_ref[...]
    seg = lax.broadcasted_iota(jnp.int32, (NSEG, D), 0)
    tot = jnp.zeros((NSEG, D), jnp.float32)
    for r in range(8):
        row = jnp.broadcast_to(blk[r:r + 1, :], (NSEG, D))
        tot = tot + (row == seg).astype(jnp.float32)
    acc_ref[...] += tot

    @pl.when(i == BC_GRID - 1)
    def _():
        cnt_ref[...] = acc_ref[...]


def _tc_finish(sums_ref, cnts_ref, out_ref):
    s = sums_ref[0] + sums_ref[1]
    c = jnp.sum(cnts_ref[...], axis=1, keepdims=True)
    out_ref[...] = s / jnp.maximum(c, 1.0)


def kernel(x, batch_index):
    bi = batch_index.astype(jnp.int32)
    mesh = plsc.VectorSubcoreMesh(core_axis_name="c", subcore_axis_name="s")

    bi2d = jnp.pad(bi, (0, 8 * BC_COLS - N_ROWS),
                   constant_values=NSEG).reshape(8, BC_COLS)
    cnts = pl.pallas_call(
        _tc_bincount,
        grid=(BC_GRID,),
        in_specs=[pl.BlockSpec((8, 128), lambda i: (0, i))],
        out_specs=pl.BlockSpec((NSEG, D), lambda i: (0, 0)),
        out_shape=jax.ShapeDtypeStruct((NSEG, D), jnp.float32),
        scratch_shapes=[pltpu.VMEM((NSEG, D), jnp.float32)],
    )(bi2d)

    @functools.partial(
        pl.kernel,
        mesh=mesh,
        out_type=jax.ShapeDtypeStruct((NC, NSEG, D), jnp.float32),
        scratch_types=(
            [pltpu.VMEM((G, D), jnp.float32) for _ in range(NBUF)]
            + [pltpu.VMEM((G,), jnp.int32) for _ in range(NBUF)]
            + [
                pltpu.VMEM((8, D), jnp.float32),
                pltpu.VMEM((8,), jnp.int32),
                pltpu.VMEM((4, D), jnp.float32),
                pltpu.VMEM_SHARED((NSEG, D), jnp.float32),
            ]
            + [pltpu.SemaphoreType.DMA for _ in range(3 * NBUF)]
        ),
    )
    def sc_seg(x_hbm, bi_hbm, sums_out, *refs):
        rows_b = refs[0:NBUF]
        idx_b = refs[NBUF:2 * NBUF]
        rows8_v, idx8_v, z_v, sums_sh = refs[2 * NBUF:2 * NBUF + 4]
        sem_gr = refs[2 * NBUF + 4:2 * NBUF + 4 + NBUF]
        sem_gi = refs[2 * NBUF + 4 + NBUF:2 * NBUF + 4 + 2 * NBUF]
        sem_s = refs[2 * NBUF + 4 + 2 * NBUF:]

        c = lax.axis_index("c")
        s = lax.axis_index("s")
        wid = c * NS + s

        zero16 = jnp.zeros((16,), jnp.float32)
        for r in range(4):
            for j in range(D // 16):
                z_v[r, pl.ds(j * 16, 16)] = zero16

        pltpu.sync_copy(z_v, sums_sh.at[pl.ds(s * 4, 4)])
        plsc.subcore_barrier()

        start = (wid * ROWS_PER_W) & -8
        end = jnp.where(wid == NW - 1, N_ROWS, ((wid + 1) * ROWS_PER_W) & -8)
        tail0 = start + N_BIG * G
        n_tail = (end - tail0) // 8

        gathers = {}
        scatters = {}

        def issue_gather(g):
            b = g % NBUF
            off = pl.multiple_of(start + g * G, 8)
            gathers[g] = (
                pltpu.async_copy(x_hbm.at[pl.ds(off, G)], rows_b[b], sem_gr[b]),
                pltpu.async_copy(bi_hbm.at[pl.ds(off, G)], idx_b[b], sem_gi[b]),
            )

        for g in range(AHEAD):
            issue_gather(g)
        for g in range(N_BIG):
            b = g % NBUF
            for d in gathers.pop(g):
                d.wait()
            scatters[g] = pltpu.async_copy(
                rows_b[b], sums_sh.at[idx_b[b]], sem_s[b], add=True)
            if g + AHEAD < N_BIG:
                prev = g + AHEAD - NBUF
                if prev >= 0:
                    scatters.pop(prev).wait()
                issue_gather(g + AHEAD)
        for g in sorted(scatters):
            scatters.pop(g).wait()

        def tail_body(t, carry):
            off = pl.multiple_of(tail0 + t * 8, 8)
            pltpu.sync_copy(x_hbm.at[pl.ds(off, 8)], rows8_v)
            pltpu.sync_copy(bi_hbm.at[pl.ds(off, 8)], idx8_v)
            pltpu.sync_copy(rows8_v, sums_sh.at[idx8_v], add=True)
            return carry

        lax.fori_loop(0, n_tail, tail_body, None)

        plsc.subcore_barrier()

        @pl.when(s == 0)
        def _():
            pltpu.sync_copy(sums_sh, sums_out.at[c])

    sums = sc_seg(x, bi)

    out = pl.pallas_call(
        _tc_finish,
        out_shape=jax.ShapeDtypeStruct((NSEG, D), jnp.float32),
    )(sums, cnts)
    return out
